# Initial kernel scaffold; baseline (speedup 1.0000x reference)
import jax
import jax.numpy as jnp
from jax.experimental import pallas as pl


def kernel(x, edge_index, root_mask, W_embed, b_embed, Ws, bs, ln_gamma, ln_beta, W_out, b_out):
    raise NotImplementedError("write your pallas kernel here")


# speedup vs baseline: 6.6680x; 6.6680x over previous
import functools

import jax
import jax.numpy as jnp
from jax import lax
from jax.experimental import pallas as pl
from jax.experimental.pallas import tpu as pltpu
from jax.experimental.pallas import tpu_sc as plsc

_N = 10000
_E = 160000
_H = 256
_HH = 128
_OUT = 128
_DEPTH = 3
_NROOTS = 1024

_RBLK = 1000
_NC = 2
_NS = 16
_K = 80
_EPT = _E // _NS
_NCHUNK = _EPT // _K
_RQ = 624
_RREM = _N - _NS * _RQ
_RPW = _NROOTS // (_NC * _NS)


def _matmul_bias_body(x_ref, w_ref, b_ref, o_ref):
    o_ref[...] = (
        jnp.dot(x_ref[...], w_ref[...], preferred_element_type=jnp.float32)
        + b_ref[...]
    )


def _embed_call(x, W, b):
    return pl.pallas_call(
        _matmul_bias_body,
        grid=(_N // _RBLK,),
        in_specs=[
            pl.BlockSpec((_RBLK, _H), lambda i: (i, 0)),
            pl.BlockSpec((_H, _H), lambda i: (0, 0)),
            pl.BlockSpec((1, _H), lambda i: (0, 0)),
        ],
        out_specs=pl.BlockSpec((_RBLK, _H), lambda i: (i, 0)),
        out_shape=jax.ShapeDtypeStruct((_N, _H), jnp.float32),
    )(x, W, b)


def _hw_body(h_ref, w_ref, deg_ref, o_ref):
    dinv = lax.rsqrt(deg_ref[...] + 1.0)
    o_ref[0] = (
        jnp.dot(h_ref[...], w_ref[...], preferred_element_type=jnp.float32)
        * dinv
    )


def _hw_call(h, W, deg):
    return pl.pallas_call(
        _hw_body,
        grid=(_N // _RBLK, _NC),
        in_specs=[
            pl.BlockSpec((_RBLK, _H), lambda i, j: (i, 0)),
            pl.BlockSpec((_H, _HH), lambda i, j: (0, j)),
            pl.BlockSpec((_RBLK, 1), lambda i, j: (i, 0)),
        ],
        out_specs=pl.BlockSpec((1, _RBLK, _HH), lambda i, j: (j, i, 0)),
        out_shape=jax.ShapeDtypeStruct((_NC, _N, _HH), jnp.float32),
    )(h, W, deg)


def _ln_body(h_ref, p_ref, hw_ref, deg_ref, b_ref, g_ref, be_ref, o_ref):
    dinv = lax.rsqrt(deg_ref[...] + 1.0)
    p = jnp.concatenate([p_ref[0], p_ref[1]], axis=1)
    hwp = jnp.concatenate([hw_ref[0], hw_ref[1]], axis=1)
    t = h_ref[...] + (p + hwp) * dinv + b_ref[...]
    mu = jnp.mean(t, axis=1, keepdims=True)
    var = jnp.mean((t - mu) * (t - mu), axis=1, keepdims=True)
    o_ref[...] = (t - mu) * lax.rsqrt(var + 1e-5) * g_ref[...] + be_ref[...]


def _ln_call(h, p2, hw2, deg, b, gamma, beta):
    return pl.pallas_call(
        _ln_body,
        grid=(_N // _RBLK,),
        in_specs=[
            pl.BlockSpec((_RBLK, _H), lambda i: (i, 0)),
            pl.BlockSpec((_NC, _RBLK, _HH), lambda i: (0, i, 0)),
            pl.BlockSpec((_NC, _RBLK, _HH), lambda i: (0, i, 0)),
            pl.BlockSpec((_RBLK, 1), lambda i: (i, 0)),
            pl.BlockSpec((1, _H), lambda i: (0, 0)),
            pl.BlockSpec((1, _H), lambda i: (0, 0)),
            pl.BlockSpec((1, _H), lambda i: (0, 0)),
        ],
        out_specs=pl.BlockSpec((_RBLK, _H), lambda i: (i, 0)),
        out_shape=jax.ShapeDtypeStruct((_N, _H), jnp.float32),
    )(h, p2, hw2, deg, b, gamma, beta)


def _out_call(hr, W, b):
    return pl.pallas_call(
        _matmul_bias_body,
        in_specs=[
            pl.BlockSpec((_NROOTS, _H), lambda: (0, 0)),
            pl.BlockSpec((_H, _OUT), lambda: (0, 0)),
            pl.BlockSpec((1, _OUT), lambda: (0, 0)),
        ],
        out_specs=pl.BlockSpec((_NROOTS, _OUT), lambda: (0, 0)),
        out_shape=jax.ShapeDtypeStruct((_NROOTS, _OUT), jnp.float32),
    )(hr, W, b)


@functools.lru_cache(maxsize=1)
def _build_sc_kernels():
    mesh = plsc.VectorSubcoreMesh(
        core_axis_name="c", subcore_axis_name="s",
        num_cores=_NC, num_subcores=_NS,
    )

    def _copy_rows(s, src, dst, src_off=0, dst_off=0):
        pltpu.sync_copy(
            src.at[pl.ds(src_off + s * _RQ, _RQ)],
            dst.at[pl.ds(dst_off + s * _RQ, _RQ)],
        )

        @pl.when(s == _NS - 1)
        def _():
            pltpu.sync_copy(
                src.at[pl.ds(src_off + _NS * _RQ, _RREM)],
                dst.at[pl.ds(dst_off + _NS * _RQ, _RREM)],
            )

    @functools.partial(
        pl.kernel,
        out_type=jax.ShapeDtypeStruct((_N, 16), jnp.float32),
        mesh=mesh,
        scratch_types=[
            pltpu.VMEM((_K,), jnp.int32),
            pltpu.VMEM((_K, 16), jnp.float32),
            pltpu.VMEM_SHARED((_N, 16), jnp.float32),
        ],
    )
    def deg_kernel(d_hbm, zeros_hbm, out_hbm, didx_v, ones_v, table_sh):
        c = lax.axis_index("c")
        s = lax.axis_index("s")

        @pl.when(c == 0)
        def _():
            _copy_rows(s, zeros_hbm, table_sh)

            def _fill(i, carry):
                ones_v[i] = jnp.ones((16,), jnp.float32)
                return carry

            lax.fori_loop(0, _K, _fill, 0)
            plsc.subcore_barrier()

            def _chunk(j, carry):
                base = s * _EPT + j * _K
                pltpu.sync_copy(d_hbm.at[pl.ds(base, _K)], didx_v)
                pltpu.sync_copy(ones_v, table_sh.at[didx_v], add=True)
                return carry

            lax.fori_loop(0, _NCHUNK, _chunk, 0)
            plsc.subcore_barrier()
            _copy_rows(s, table_sh, out_hbm)

    @functools.partial(
        pl.kernel,
        out_type=jax.ShapeDtypeStruct((_NC * _N, _HH), jnp.float32),
        mesh=mesh,
        scratch_types=[
            pltpu.VMEM((_K,), jnp.int32),
            pltpu.VMEM((_K,), jnp.int32),
            pltpu.VMEM((_K, _HH), jnp.float32),
            pltpu.VMEM_SHARED((_N, _HH), jnp.float32),
            pltpu.SemaphoreType.DMA,
        ],
    )
    def agg_kernel(s_hbm, d_hbm, hw_hbm, zeros_hbm, out_hbm,
                   sidx_v, didx_v, rows_v, acc_sh, sem):
        c = lax.axis_index("c")
        s = lax.axis_index("s")
        rowoff = c * _N

        _copy_rows(s, zeros_hbm, acc_sh)
        plsc.subcore_barrier()

        def _chunk(j, carry):
            base = s * _EPT + j * _K
            pltpu.sync_copy(s_hbm.at[pl.ds(base, _K)], sidx_v)
            pltpu.sync_copy(d_hbm.at[pl.ds(base, _K)], didx_v)

            def _addoff(i, carry2):
                sidx_v[pl.ds(i * 16, 16)] = sidx_v[pl.ds(i * 16, 16)] + rowoff
                return carry2

            lax.fori_loop(0, _K // 16, _addoff, 0)
            pltpu.async_copy(hw_hbm.at[sidx_v], rows_v, sem).wait()
            pltpu.sync_copy(rows_v, acc_sh.at[didx_v], add=True)
            return carry

        lax.fori_loop(0, _NCHUNK, _chunk, 0)
        plsc.subcore_barrier()
        _copy_rows(s, acc_sh, out_hbm, dst_off=rowoff)

    @functools.partial(
        pl.kernel,
        out_type=jax.ShapeDtypeStruct((_NROOTS, _H), jnp.float32),
        mesh=mesh,
        scratch_types=[
            pltpu.VMEM((_RPW,), jnp.int32),
            pltpu.VMEM((_RPW, _H), jnp.float32),
            pltpu.SemaphoreType.DMA,
        ],
    )
    def root_gather(h_hbm, roots_hbm, out_hbm, idx_v, rows_v, sem):
        wid = lax.axis_index("s") * _NC + lax.axis_index("c")
        base = wid * _RPW
        pltpu.sync_copy(roots_hbm.at[pl.ds(base, _RPW)], idx_v)
        pltpu.async_copy(h_hbm.at[idx_v], rows_v, sem).wait()
        pltpu.sync_copy(rows_v, out_hbm.at[pl.ds(base, _RPW)])

    return deg_kernel, agg_kernel, root_gather


def kernel(x, edge_index, root_mask, W_embed, b_embed, Ws, bs,
           ln_gamma, ln_beta, W_out, b_out):
    deg_kernel, agg_kernel, root_gather = _build_sc_kernels()
    src = edge_index[0]
    dst = edge_index[1]
    zeros16 = jnp.zeros((_N, 16), jnp.float32)
    zeros128 = jnp.zeros((_N, _HH), jnp.float32)

    deg = deg_kernel(dst, zeros16)[:, :1]
    h = _embed_call(x, W_embed, b_embed.reshape(1, _H))
    for i in range(_DEPTH):
        hw2 = _hw_call(h, Ws[i], deg)
        p = agg_kernel(src, dst, hw2.reshape(_NC * _N, _HH), zeros128)
        h = _ln_call(
            h, p.reshape(_NC, _N, _HH), hw2, deg,
            bs[i].reshape(1, _H),
            ln_gamma[i].reshape(1, _H),
            ln_beta[i].reshape(1, _H),
        )
    hr = root_gather(h, root_mask)
    return _out_call(hr, W_out, b_out.reshape(1, _OUT))


# speedup vs baseline: 9.0433x; 1.3562x over previous
import functools

import jax
import jax.numpy as jnp
from jax import lax
from jax.experimental import pallas as pl
from jax.experimental.pallas import tpu as pltpu
from jax.experimental.pallas import tpu_sc as plsc

_N = 10000
_E = 160000
_H = 256
_HH = 128
_OUT = 128
_DEPTH = 3
_NROOTS = 1024

_RBLK = 1000
_NC = 2
_NS = 16
_K = 80
_EPT = _E // _NS
_NCHUNK = _EPT // _K
_RQ = 624
_RREM = _N - _NS * _RQ
_RPW = _NROOTS // (_NC * _NS)


def _matmul_bias_body(x_ref, w_ref, b_ref, o_ref):
    o_ref[...] = (
        jnp.dot(x_ref[...], w_ref[...], preferred_element_type=jnp.float32)
        + b_ref[...]
    )


def _embed_call(x, W, b):
    return pl.pallas_call(
        _matmul_bias_body,
        grid=(_N // _RBLK,),
        in_specs=[
            pl.BlockSpec((_RBLK, _H), lambda i: (i, 0)),
            pl.BlockSpec((_H, _H), lambda i: (0, 0)),
            pl.BlockSpec((1, _H), lambda i: (0, 0)),
        ],
        out_specs=pl.BlockSpec((_RBLK, _H), lambda i: (i, 0)),
        out_shape=jax.ShapeDtypeStruct((_N, _H), jnp.float32),
    )(x, W, b)


def _hw_body(h_ref, w_ref, deg_ref, o_ref):
    dinv = lax.rsqrt(deg_ref[...] + 1.0)
    o_ref[0] = (
        jnp.dot(h_ref[...], w_ref[...], preferred_element_type=jnp.float32)
        * dinv
    )


def _hw_call(h, W, deg):
    return pl.pallas_call(
        _hw_body,
        grid=(_N // _RBLK, _NC),
        in_specs=[
            pl.BlockSpec((_RBLK, _H), lambda i, j: (i, 0)),
            pl.BlockSpec((_H, _HH), lambda i, j: (0, j)),
            pl.BlockSpec((_RBLK, 1), lambda i, j: (i, 0)),
        ],
        out_specs=pl.BlockSpec((1, _RBLK, _HH), lambda i, j: (j, i, 0)),
        out_shape=jax.ShapeDtypeStruct((_NC, _N, _HH), jnp.float32),
    )(h, W, deg)


def _ln_body(h_ref, p_ref, hw_ref, deg_ref, b_ref, g_ref, be_ref, o_ref):
    dinv = lax.rsqrt(deg_ref[...] + 1.0)
    p = jnp.concatenate([p_ref[0], p_ref[1]], axis=1)
    hwp = jnp.concatenate([hw_ref[0], hw_ref[1]], axis=1)
    t = h_ref[...] + (p + hwp) * dinv + b_ref[...]
    mu = jnp.mean(t, axis=1, keepdims=True)
    var = jnp.mean((t - mu) * (t - mu), axis=1, keepdims=True)
    o_ref[...] = (t - mu) * lax.rsqrt(var + 1e-5) * g_ref[...] + be_ref[...]


def _ln_call(h, p2, hw2, deg, b, gamma, beta):
    return pl.pallas_call(
        _ln_body,
        grid=(_N // _RBLK,),
        in_specs=[
            pl.BlockSpec((_RBLK, _H), lambda i: (i, 0)),
            pl.BlockSpec((_NC, _RBLK, _HH), lambda i: (0, i, 0)),
            pl.BlockSpec((_NC, _RBLK, _HH), lambda i: (0, i, 0)),
            pl.BlockSpec((_RBLK, 1), lambda i: (i, 0)),
            pl.BlockSpec((1, _H), lambda i: (0, 0)),
            pl.BlockSpec((1, _H), lambda i: (0, 0)),
            pl.BlockSpec((1, _H), lambda i: (0, 0)),
        ],
        out_specs=pl.BlockSpec((_RBLK, _H), lambda i: (i, 0)),
        out_shape=jax.ShapeDtypeStruct((_N, _H), jnp.float32),
    )(h, p2, hw2, deg, b, gamma, beta)


def _out_call(hr, W, b):
    return pl.pallas_call(
        _matmul_bias_body,
        in_specs=[
            pl.BlockSpec((_NROOTS, _H), lambda: (0, 0)),
            pl.BlockSpec((_H, _OUT), lambda: (0, 0)),
            pl.BlockSpec((1, _OUT), lambda: (0, 0)),
        ],
        out_specs=pl.BlockSpec((_NROOTS, _OUT), lambda: (0, 0)),
        out_shape=jax.ShapeDtypeStruct((_NROOTS, _OUT), jnp.float32),
    )(hr, W, b)


@functools.lru_cache(maxsize=1)
def _build_sc_kernels():
    mesh = plsc.VectorSubcoreMesh(
        core_axis_name="c", subcore_axis_name="s",
        num_cores=_NC, num_subcores=_NS,
    )

    def _copy_rows(s, src, dst, src_off=0, dst_off=0):
        pltpu.sync_copy(
            src.at[pl.ds(src_off + s * _RQ, _RQ)],
            dst.at[pl.ds(dst_off + s * _RQ, _RQ)],
        )

        @pl.when(s == _NS - 1)
        def _():
            pltpu.sync_copy(
                src.at[pl.ds(src_off + _NS * _RQ, _RREM)],
                dst.at[pl.ds(dst_off + _NS * _RQ, _RREM)],
            )

    @functools.partial(
        pl.kernel,
        out_type=jax.ShapeDtypeStruct((_N, 16), jnp.float32),
        mesh=mesh,
        scratch_types=[
            pltpu.VMEM((_K,), jnp.int32),
            pltpu.VMEM((_K, 16), jnp.float32),
            pltpu.VMEM_SHARED((_N, 16), jnp.float32),
        ],
    )
    def deg_kernel(d_hbm, zeros_hbm, out_hbm, didx_v, ones_v, table_sh):
        c = lax.axis_index("c")
        s = lax.axis_index("s")

        @pl.when(c == 0)
        def _():
            _copy_rows(s, zeros_hbm, table_sh)

            def _fill(i, carry):
                ones_v[i] = jnp.ones((16,), jnp.float32)
                return carry

            lax.fori_loop(0, _K, _fill, 0)
            plsc.subcore_barrier()

            def _chunk(j, carry):
                base = s * _EPT + j * _K
                pltpu.sync_copy(d_hbm.at[pl.ds(base, _K)], didx_v)
                pltpu.sync_copy(ones_v, table_sh.at[didx_v], add=True)
                return carry

            lax.fori_loop(0, _NCHUNK, _chunk, 0)
            plsc.subcore_barrier()
            _copy_rows(s, table_sh, out_hbm)

    @functools.partial(
        pl.kernel,
        out_type=jax.ShapeDtypeStruct((_NC * _N, _HH), jnp.float32),
        mesh=mesh,
        scratch_types=[
            pltpu.VMEM((_K,), jnp.int32),
            pltpu.VMEM((_K,), jnp.int32),
            pltpu.VMEM((_K,), jnp.int32),
            pltpu.VMEM((_K,), jnp.int32),
            pltpu.VMEM((_K, _HH), jnp.float32),
            pltpu.VMEM((_K, _HH), jnp.float32),
            pltpu.VMEM_SHARED((_N, _HH), jnp.float32),
            pltpu.SemaphoreType.DMA,
            pltpu.SemaphoreType.DMA,
        ],
    )
    def agg_kernel(s_hbm, d_hbm, hw_hbm, zeros_hbm, out_hbm,
                   sidx_a, sidx_b, didx_a, didx_b, rows_a, rows_b, acc_sh,
                   sem_a, sem_b):
        c = lax.axis_index("c")
        s = lax.axis_index("s")
        rowoff = c * _N

        def _sslice(j):
            return s_hbm.at[pl.ds((c * _NS + s) * _EPT + j * _K, _K)]

        def _dslice(j):
            return d_hbm.at[pl.ds(s * _EPT + j * _K, _K)]

        _copy_rows(s, zeros_hbm, acc_sh)
        plsc.subcore_barrier()

        pltpu.sync_copy(_sslice(0), sidx_a)
        pltpu.async_copy(hw_hbm.at[sidx_a], rows_a, sem_a).wait()
        pltpu.sync_copy(_dslice(0), didx_a)

        def _pair(t, carry):
            j0 = 2 * t
            pltpu.sync_copy(_sslice(j0 + 1), sidx_b)
            cp_b = pltpu.async_copy(hw_hbm.at[sidx_b], rows_b, sem_b)
            pltpu.sync_copy(_dslice(j0 + 1), didx_b)
            pltpu.sync_copy(rows_a, acc_sh.at[didx_a], add=True)
            cp_b.wait()
            pltpu.sync_copy(_sslice(j0 + 2), sidx_a)
            cp_a = pltpu.async_copy(hw_hbm.at[sidx_a], rows_a, sem_a)
            pltpu.sync_copy(_dslice(j0 + 2), didx_a)
            pltpu.sync_copy(rows_b, acc_sh.at[didx_b], add=True)
            cp_a.wait()
            return carry

        lax.fori_loop(0, (_NCHUNK - 1) // 2, _pair, 0)
        pltpu.sync_copy(rows_a, acc_sh.at[didx_a], add=True)

        plsc.subcore_barrier()
        _copy_rows(s, acc_sh, out_hbm, dst_off=rowoff)

    @functools.partial(
        pl.kernel,
        out_type=jax.ShapeDtypeStruct((_NROOTS, _H), jnp.float32),
        mesh=mesh,
        scratch_types=[
            pltpu.VMEM((_RPW,), jnp.int32),
            pltpu.VMEM((8, _H), jnp.float32),
            pltpu.SemaphoreType.DMA,
        ],
    )
    def root_gather(h_hbm, roots_hbm, out_hbm, idx_v, rows_v, sem):
        wid = lax.axis_index("s") * _NC + lax.axis_index("c")
        base = wid * _RPW
        pltpu.sync_copy(roots_hbm.at[pl.ds(base, _RPW)], idx_v)

        def _step(t, carry):
            pltpu.async_copy(h_hbm.at[idx_v.at[pl.ds(t * 8, 8)]],
                             rows_v, sem).wait()
            pltpu.sync_copy(rows_v, out_hbm.at[pl.ds(base + t * 8, 8)])
            return carry

        lax.fori_loop(0, _RPW // 8, _step, 0)

    return deg_kernel, agg_kernel, root_gather


def kernel(x, edge_index, root_mask, W_embed, b_embed, Ws, bs,
           ln_gamma, ln_beta, W_out, b_out):
    deg_kernel, agg_kernel, root_gather = _build_sc_kernels()
    src = edge_index[0]
    dst = edge_index[1]
    src2 = jnp.concatenate([src, src + _N]).reshape(_NC * _NS * _EPT)
    zeros16 = jnp.zeros((_N, 16), jnp.float32)
    zeros128 = jnp.zeros((_N, _HH), jnp.float32)

    deg = deg_kernel(dst, zeros16)[:, :1]
    h = _embed_call(x, W_embed, b_embed.reshape(1, _H))
    for i in range(_DEPTH):
        hw2 = _hw_call(h, Ws[i], deg)
        p = agg_kernel(src2, dst, hw2.reshape(_NC * _N, _HH), zeros128)
        h = _ln_call(
            h, p.reshape(_NC, _N, _HH), hw2, deg,
            bs[i].reshape(1, _H),
            ln_gamma[i].reshape(1, _H),
            ln_beta[i].reshape(1, _H),
        )
    hr = root_gather(h, root_mask)
    return _out_call(hr, W_out, b_out.reshape(1, _OUT))
